# baseline (speedup 1.0000x reference)
import functools

import jax
import jax.numpy as jnp
from jax import lax
from jax.experimental import pallas as pl
from jax.experimental.pallas import tpu as pltpu
from jax.experimental.pallas import tpu_sc as plsc

_BINS = [-3.0, -2.8, -2.6, -2.4, -2.2, -2.0, -1.8, -1.6, -1.4, -1.2, -1.0,
         -0.8, -0.6, -0.4, -0.2, 0.0, 0.2, 0.4, 0.6, 0.8, 1.0, 1.2, 1.4,
         1.6, 1.8, 2.0, 2.2, 2.4, 2.6, 2.8, 3.0]

_ROWS = 16384
_COLS = 4096
_NC = 2
_NS = 16
_NW = _NC * _NS
_RW = _ROWS // _NW
_CR = 8
_NCH = _RW // _CR
_NBUF = 3


def _compute(buf, bins_v):
    for r in range(_CR):
        @plsc.parallel_loop(0, _COLS, step=16, unroll=8)
        def vec_body(c):
            v = plsc.bitcast(buf[r, pl.ds(c, 16)], jnp.float32)
            t = v * 5.0 + 15.5
            t = jnp.minimum(jnp.maximum(t, 0.5), 30.5)
            k = t.astype(jnp.int32)
            b = plsc.load_gather(bins_v, [k])
            buf[r, pl.ds(c, 16)] = k + jnp.where(b <= v, 1, 0)


def _body(x_hbm, bins_hbm, out_hbm, bins_v, b0, b1, b2,
          si0, si1, si2, so0, so1, so2):
    wid = lax.axis_index("s") * _NC + lax.axis_index("c")
    row_w = wid * _RW
    bufs = (b0, b1, b2)
    sis, sos = (si0, si1, si2), (so0, so1, so2)
    pltpu.sync_copy(bins_hbm, bins_v)

    def rows(ci):
        return pl.ds(row_w + ci * _CR, _CR)

    def step(ci, b):
        pltpu.make_async_copy(x_hbm.at[rows(ci)], bufs[b], sis[b]).wait()
        _compute(bufs[b], bins_v)
        pltpu.async_copy(bufs[b], out_hbm.at[rows(ci)], sos[b])

        nb = (b + 2) % _NBUF

        @pl.when(ci + 2 < _NCH)
        def _start_next_in():
            @pl.when(ci >= 1)
            def _wait_prev_out():
                pltpu.make_async_copy(
                    bufs[nb], out_hbm.at[rows(ci - 1)], sos[nb]).wait()

            pltpu.async_copy(x_hbm.at[rows(ci + 2)], bufs[nb], sis[nb])

    for ci in range(2):
        pltpu.async_copy(x_hbm.at[rows(ci)], bufs[ci], sis[ci])

    n_full = _NCH // _NBUF

    @pl.loop(0, n_full)
    def group(g):
        for b in range(_NBUF):
            step(g * _NBUF + b, b)

    for ci in range(n_full * _NBUF, _NCH):
        step(ci, ci % _NBUF)

    for ci in range(_NCH - _NBUF, _NCH):
        b = ci % _NBUF
        pltpu.make_async_copy(bufs[b], out_hbm.at[rows(ci)], sos[b]).wait()


@jax.jit
def kernel(inputs):
    xi = lax.bitcast_convert_type(inputs, jnp.int32)
    bins = jnp.asarray(_BINS + [3.0], dtype=jnp.float32)
    mesh = plsc.VectorSubcoreMesh(core_axis_name="c", subcore_axis_name="s")
    run = functools.partial(
        pl.kernel,
        out_type=jax.ShapeDtypeStruct((_ROWS, _COLS), jnp.int32),
        mesh=mesh,
        compiler_params=pltpu.CompilerParams(needs_layout_passes=False),
        scratch_types=[
            pltpu.VMEM((32,), jnp.float32),
            pltpu.VMEM((_CR, _COLS), jnp.int32),
            pltpu.VMEM((_CR, _COLS), jnp.int32),
            pltpu.VMEM((_CR, _COLS), jnp.int32),
            pltpu.SemaphoreType.DMA,
            pltpu.SemaphoreType.DMA,
            pltpu.SemaphoreType.DMA,
            pltpu.SemaphoreType.DMA,
            pltpu.SemaphoreType.DMA,
            pltpu.SemaphoreType.DMA,
        ],
    )(_body)
    return run(xi, bins)


# speedup vs baseline: 1.4824x; 1.4824x over previous
import functools

import jax
import jax.numpy as jnp
from jax import lax
from jax.experimental import pallas as pl
from jax.experimental.pallas import tpu as pltpu
from jax.experimental.pallas import tpu_sc as plsc

_BINS = [-3.0, -2.8, -2.6, -2.4, -2.2, -2.0, -1.8, -1.6, -1.4, -1.2, -1.0,
         -0.8, -0.6, -0.4, -0.2, 0.0, 0.2, 0.4, 0.6, 0.8, 1.0, 1.2, 1.4,
         1.6, 1.8, 2.0, 2.2, 2.4, 2.6, 2.8, 3.0]

_ROWS = 16384
_COLS = 4096
_NC = 2
_NS = 16
_NW = _NC * _NS
_RW = _ROWS // _NW
_CR = 8
_CC = 2048
_NG = _RW // _CR


def _compute(inbuf, outbuf, bins_v):
    for r in range(_CR):
        @plsc.parallel_loop(0, _CC, step=16, unroll=8)
        def vec_body(c):
            v = inbuf[r, pl.ds(c, 16)]
            t = v * 5.0 + 15.5
            t = jnp.minimum(jnp.maximum(t, 0.5), 30.5)
            k = t.astype(jnp.int32)
            b = plsc.load_gather(bins_v, [k])
            outbuf[r, pl.ds(c, 16)] = k + jnp.where(b <= v, 1, 0)


def _body(x_hbm, bins_hbm, out_hbm, bins_v, in0, in1, out0, out1,
          si0, si1, so0, so1):
    wid = lax.axis_index("s") * _NC + lax.axis_index("c")
    row_w = wid * _RW
    ins, outs = (in0, in1), (out0, out1)
    sis, sos = (si0, si1), (so0, so1)

    for b in range(2):
        pltpu.async_copy(
            x_hbm.at[pl.ds(row_w, _CR), pl.ds(b * _CC, _CC)], ins[b], sis[b])
    pltpu.sync_copy(bins_hbm, bins_v)

    @pl.loop(0, _NG)
    def group(g):
        r0 = row_w + g * _CR
        for b in range(2):
            c0 = b * _CC
            pltpu.make_async_copy(
                x_hbm.at[pl.ds(r0, _CR), pl.ds(c0, _CC)], ins[b],
                sis[b]).wait()

            @pl.when(g >= 1)
            def _wait_prev_out():
                pltpu.make_async_copy(
                    outs[b], out_hbm.at[pl.ds(r0 - _CR, _CR), pl.ds(c0, _CC)],
                    sos[b]).wait()

            _compute(ins[b], outs[b], bins_v)

            @pl.when(g + 1 < _NG)
            def _start_next_in():
                pltpu.async_copy(
                    x_hbm.at[pl.ds(r0 + _CR, _CR), pl.ds(c0, _CC)],
                    ins[b], sis[b])

            pltpu.async_copy(
                outs[b], out_hbm.at[pl.ds(r0, _CR), pl.ds(c0, _CC)], sos[b])

    last_r0 = row_w + (_NG - 1) * _CR
    for b in range(2):
        pltpu.make_async_copy(
            outs[b], out_hbm.at[pl.ds(last_r0, _CR), pl.ds(b * _CC, _CC)],
            sos[b]).wait()


@jax.jit
def kernel(inputs):
    bins = jnp.asarray(_BINS + [3.0], dtype=jnp.float32)
    mesh = plsc.VectorSubcoreMesh(core_axis_name="c", subcore_axis_name="s")
    run = functools.partial(
        pl.kernel,
        out_type=jax.ShapeDtypeStruct((_ROWS, _COLS), jnp.int32),
        mesh=mesh,
        compiler_params=pltpu.CompilerParams(needs_layout_passes=False),
        scratch_types=[
            pltpu.VMEM((32,), jnp.float32),
            pltpu.VMEM((_CR, _CC), jnp.float32),
            pltpu.VMEM((_CR, _CC), jnp.float32),
            pltpu.VMEM((_CR, _CC), jnp.int32),
            pltpu.VMEM((_CR, _CC), jnp.int32),
            pltpu.SemaphoreType.DMA,
            pltpu.SemaphoreType.DMA,
            pltpu.SemaphoreType.DMA,
            pltpu.SemaphoreType.DMA,
        ],
    )(_body)
    return run(inputs, bins)
